# baseline (speedup 1.0000x reference)
import jax
import jax.numpy as jnp
from jax import lax
from jax.experimental import pallas as pl
from jax.experimental.pallas import tpu as pltpu
from jax.experimental.pallas import tpu_sc as plsc

_B = 1024
_C = 100000
_B_TC = 768
_BR = 64

_NC = 2
_NS = 16
_NW = _NC * _NS
_SC_ROWS = _B - _B_TC
_RPW = _SC_ROWS // _NW
_NT = 781
_TAIL0 = _NT * 128
_TAIL_W = _C - _TAIL0
_NBUF = 8


def _tc_body(y_ref, x_ref, o_ref):
    x = x_ref[...]
    yv = y_ref[...]
    c = x.shape[1]
    cols = jax.lax.broadcasted_iota(jnp.int32, x.shape, 1)
    oy = jnp.sum(jnp.where(cols == yv, x, 0.0), axis=1, keepdims=True)
    s = jnp.sum(jnp.maximum(x - (oy - 1.0), 0.0), axis=1, keepdims=True)
    o_ref[...] = (s - 1.0) / c


def _tailgather_body(y_ref, x_ref, o_ref):
    x = x_ref[...]
    yv = y_ref[...]
    cols = jax.lax.broadcasted_iota(jnp.int32, x.shape, 1) + _TAIL0
    o_ref[...] = jnp.sum(jnp.where(cols == yv, x, 0.0), axis=1,
                         keepdims=True)


def _tail_body(oy_ref, sp_ref, x_ref, o_ref):
    x = x_ref[...]
    t = oy_ref[...] - 1.0
    part = jnp.sum(jnp.maximum(x - t, 0.0), axis=1, keepdims=True)
    o_ref[...] = (sp_ref[...] + part - 1.0) / _C


def _sc_body(x_hbm, y_hbm, oyt_hbm, sp_hbm, oyf_hbm,
             y_v, oyt_v, oyb, buf3, res_v, oyr_v, shared_sp, shared_oy, sem):
    cid = lax.axis_index("c")
    sid = lax.axis_index("s")
    wid = cid * _NS + sid
    base = pl.multiple_of(wid * _RPW, 8)
    pltpu.sync_copy(y_hbm, y_v)
    pltpu.sync_copy(oyt_hbm, oyt_v)
    lane = lax.iota(jnp.int32, 16)
    row0 = wid * _RPW
    sel0 = row0 & 15
    b16 = pl.multiple_of((row0 >> 4) << 4, 16)
    y16 = y_v[pl.ds(pl.multiple_of(_B_TC + b16, 16), 16)]
    oyt16 = oyt_v[pl.ds(b16, 16)]

    res = jnp.zeros((16,), jnp.float32)
    oyr = jnp.zeros((16,), jnp.float32)
    for g in range(_RPW // 8):
        rbase = pl.multiple_of(base + g * 8, 8)
        ts = []
        for rr in range(8):
            lidx = sel0 + g * 8 + rr
            y_r = jnp.sum(jnp.where(lane == lidx, y16, 0))
            tile = jnp.minimum(y_r >> 7, _NT - 1)
            c0 = pl.multiple_of(tile << 7, 128)
            pltpu.sync_copy(x_hbm.at[pl.ds(rbase, 8), pl.ds(c0, 128)], oyb)
            off = jnp.minimum(y_r - (tile << 7), 127)
            v16 = oyb[rr, pl.ds(pl.multiple_of((off >> 4) << 4, 16), 16)]
            oy_dma = jnp.sum(jnp.where(lane == (off & 15), v16, 0.0))
            oy_tail = jnp.sum(jnp.where(lane == lidx, oyt16, 0.0))
            oy_r = jnp.where(y_r < _TAIL0, oy_dma, oy_tail)
            oyr = jnp.where(lane == g * 8 + rr, oy_r, oyr)
            ts.append(oy_r - 1.0)

        def tile_copy(kt, slot):
            c0 = pl.multiple_of(kt * 128, 128)
            return pltpu.make_async_copy(
                x_hbm.at[pl.ds(rbase, 8), pl.ds(c0, 128)],
                buf3.at[slot], sem.at[slot])

        for s in range(_NBUF):
            tile_copy(s, s).start()

        def batch(gb, accs):
            new = accs
            for s in range(_NBUF):
                kt = gb * _NBUF + s
                tile_copy(kt, s).wait()

                def inner(j, a, s=s):
                    return tuple(
                        a[rr] + jnp.maximum(
                            buf3[s, rr, pl.ds(j * 16, 16)] - ts[rr], 0.0)
                        for rr in range(8))

                new = lax.fori_loop(0, 8, inner, new)

                @pl.when(kt + _NBUF < _NT)
                def _():
                    tile_copy(kt + _NBUF, s).start()
            return new

        accs = tuple(jnp.zeros((16,), jnp.float32) for _ in range(8))
        accs = lax.fori_loop(0, _NT // _NBUF, batch, accs)
        nfull = (_NT // _NBUF) * _NBUF
        for s in range(_NT - nfull):
            tile_copy(nfull + s, s).wait()

            def inner_t(j, a, s=s):
                return tuple(
                    a[rr] + jnp.maximum(
                        buf3[s, rr, pl.ds(j * 16, 16)] - ts[rr], 0.0)
                    for rr in range(8))

            accs = lax.fori_loop(0, 8, inner_t, accs)

        for rr in range(8):
            s_r = jnp.sum(accs[rr])
            res = jnp.where(lane == g * 8 + rr, s_r, res)
    res_v[...] = res
    oyr_v[...] = oyr
    pltpu.sync_copy(res_v.at[pl.ds(0, _RPW)],
                    shared_sp.at[pl.ds(sid * _RPW, _RPW)])
    pltpu.sync_copy(oyr_v.at[pl.ds(0, _RPW)],
                    shared_oy.at[pl.ds(sid * _RPW, _RPW)])
    plsc.subcore_barrier()

    @pl.when(sid == 0)
    def _():
        half = _SC_ROWS // _NC
        off = pl.multiple_of(cid * half, 8)
        pltpu.sync_copy(shared_sp, sp_hbm.at[pl.ds(off, half)])
        pltpu.sync_copy(shared_oy, oyf_hbm.at[pl.ds(off, half)])


def _sc_partial(output, y1, oy_tail):
    mesh = plsc.VectorSubcoreMesh(core_axis_name="c", subcore_axis_name="s")
    f = pl.kernel(
        _sc_body,
        out_type=(jax.ShapeDtypeStruct((_SC_ROWS,), jnp.float32),
                  jax.ShapeDtypeStruct((_SC_ROWS,), jnp.float32)),
        mesh=mesh,
        scratch_types=[
            pltpu.VMEM((_B,), jnp.int32),
            pltpu.VMEM((_SC_ROWS,), jnp.float32),
            pltpu.VMEM((8, 128), jnp.float32),
            pltpu.VMEM((_NBUF, 8, 128), jnp.float32),
            pltpu.VMEM((16,), jnp.float32),
            pltpu.VMEM((16,), jnp.float32),
            pltpu.VMEM_SHARED((_SC_ROWS // _NC,), jnp.float32),
            pltpu.VMEM_SHARED((_SC_ROWS // _NC,), jnp.float32),
            pltpu.SemaphoreType.DMA((_NBUF,)),
        ],
        compiler_params=pltpu.CompilerParams(needs_layout_passes=False),
    )
    return f(output, y1, oy_tail)


def kernel(output, y):
    b, c = output.shape
    y1 = y.astype(jnp.int32)
    y2 = y1.reshape(b, 1)

    loss_tc = pl.pallas_call(
        _tc_body,
        grid=(_B_TC // _BR,),
        in_specs=[
            pl.BlockSpec((_BR, 1), lambda i: (i, 0)),
            pl.BlockSpec((_BR, c), lambda i: (i, 0)),
        ],
        out_specs=pl.BlockSpec((_BR, 1), lambda i: (i, 0)),
        out_shape=jax.ShapeDtypeStruct((_B_TC, 1), jnp.float32),
    )(y2, lax.slice(output, (0, 0), (_B_TC, _C)))

    x32 = lax.slice(output, (_B_TC, _TAIL0), (_B, _C))

    oy_tail = pl.pallas_call(
        _tailgather_body,
        grid=(1,),
        in_specs=[
            pl.BlockSpec((_SC_ROWS, 1), lambda i: (3, 0)),
            pl.BlockSpec((_SC_ROWS, _TAIL_W), lambda i: (0, 0)),
        ],
        out_specs=pl.BlockSpec((_SC_ROWS, 1), lambda i: (0, 0)),
        out_shape=jax.ShapeDtypeStruct((_SC_ROWS, 1), jnp.float32),
    )(y2, x32)

    x_sc = lax.slice(output, (_B_TC, 0), (_B, _C))
    sp, oyf = _sc_partial(x_sc, y1, oy_tail.reshape(_SC_ROWS))

    loss_tail = pl.pallas_call(
        _tail_body,
        grid=(_SC_ROWS // 64,),
        in_specs=[
            pl.BlockSpec((64, 1), lambda i: (i, 0)),
            pl.BlockSpec((64, 1), lambda i: (i, 0)),
            pl.BlockSpec((64, _TAIL_W), lambda i: (i, 0)),
        ],
        out_specs=pl.BlockSpec((64, 1), lambda i: (i, 0)),
        out_shape=jax.ShapeDtypeStruct((_SC_ROWS, 1), jnp.float32),
    )(oyf.reshape(_SC_ROWS, 1), sp.reshape(_SC_ROWS, 1), x32)

    return jnp.concatenate([loss_tc.reshape(_B_TC),
                            loss_tail.reshape(_SC_ROWS)])


# speedup vs baseline: 1.5064x; 1.5064x over previous
import jax
import jax.numpy as jnp
from jax import lax
from jax.experimental import pallas as pl
from jax.experimental.pallas import tpu as pltpu
from jax.experimental.pallas import tpu_sc as plsc

_B = 1024
_C = 100000
_B_TC = 768
_BR = 64

_NC = 2
_NS = 16
_NW = _NC * _NS
_SC_ROWS = _B - _B_TC
_RPW = _SC_ROWS // _NW
_NT = 781
_TAIL0 = _NT * 128
_TAIL_W = _C - _TAIL0
_NBUF = 8


def _tc_body(y_ref, x_ref, o_ref):
    x = x_ref[...]
    yv = y_ref[...]
    c = x.shape[1]
    cols = jax.lax.broadcasted_iota(jnp.int32, x.shape, 1)
    oy = jnp.sum(jnp.where(cols == yv, x, 0.0), axis=1, keepdims=True)
    s = jnp.sum(jnp.maximum(x - (oy - 1.0), 0.0), axis=1, keepdims=True)
    o_ref[...] = (s - 1.0) / c


def _tailgather_body(y_ref, x_ref, o_ref):
    x = x_ref[...]
    yv = y_ref[...]
    cols = jax.lax.broadcasted_iota(jnp.int32, x.shape, 1) + _TAIL0
    o_ref[...] = jnp.sum(jnp.where(cols == yv, x, 0.0), axis=1,
                         keepdims=True)


def _tail_body(oy_ref, sp_ref, x_ref, o_ref):
    x = x_ref[...]
    t = oy_ref[...] - 1.0
    part = jnp.sum(jnp.maximum(x - t, 0.0), axis=1, keepdims=True)
    o_ref[...] = (sp_ref[...] + part - 1.0) / _C


def _sc_body(x_hbm, y_hbm, oyt_hbm, sp_hbm, oyf_hbm,
             y_v, oyt_v, oyb, buf3, res_v, oyr_v, shared_sp, shared_oy, sem):
    cid = lax.axis_index("c")
    sid = lax.axis_index("s")
    wid = cid * _NS + sid
    base = pl.multiple_of(_B_TC + wid * _RPW, 8)
    pltpu.sync_copy(y_hbm, y_v)
    pltpu.sync_copy(oyt_hbm, oyt_v)
    lane = lax.iota(jnp.int32, 16)
    row0 = wid * _RPW
    sel0 = row0 & 15
    b16 = pl.multiple_of((row0 >> 4) << 4, 16)
    y16 = y_v[pl.ds(pl.multiple_of(_B_TC + b16, 16), 16)]
    oyt16 = oyt_v[pl.ds(b16, 16)]

    res = jnp.zeros((16,), jnp.float32)
    oyr = jnp.zeros((16,), jnp.float32)
    for g in range(_RPW // 8):
        rbase = pl.multiple_of(base + g * 8, 8)
        ts = []
        for rr in range(8):
            lidx = sel0 + g * 8 + rr
            y_r = jnp.sum(jnp.where(lane == lidx, y16, 0))
            tile = jnp.minimum(y_r >> 7, _NT - 1)
            c0 = pl.multiple_of(tile << 7, 128)
            pltpu.sync_copy(x_hbm.at[pl.ds(rbase, 8), pl.ds(c0, 128)], oyb)
            off = jnp.minimum(y_r - (tile << 7), 127)
            v16 = oyb[rr, pl.ds(pl.multiple_of((off >> 4) << 4, 16), 16)]
            oy_dma = jnp.sum(jnp.where(lane == (off & 15), v16, 0.0))
            oy_tail = jnp.sum(jnp.where(lane == lidx, oyt16, 0.0))
            oy_r = jnp.where(y_r < _TAIL0, oy_dma, oy_tail)
            oyr = jnp.where(lane == g * 8 + rr, oy_r, oyr)
            ts.append(oy_r - 1.0)

        def tile_copy(kt, slot):
            c0 = pl.multiple_of(kt * 128, 128)
            return pltpu.make_async_copy(
                x_hbm.at[pl.ds(rbase, 8), pl.ds(c0, 128)],
                buf3.at[slot], sem.at[slot])

        for s in range(_NBUF):
            tile_copy(s, s).start()

        def batch(gb, accs):
            new = accs
            for s in range(_NBUF):
                kt = gb * _NBUF + s
                tile_copy(kt, s).wait()

                def inner(j, a, s=s):
                    return tuple(
                        a[rr] + jnp.maximum(
                            buf3[s, rr, pl.ds(j * 16, 16)] - ts[rr], 0.0)
                        for rr in range(8))

                new = lax.fori_loop(0, 8, inner, new)

                @pl.when(kt + _NBUF < _NT)
                def _():
                    tile_copy(kt + _NBUF, s).start()
            return new

        accs = tuple(jnp.zeros((16,), jnp.float32) for _ in range(8))
        accs = lax.fori_loop(0, _NT // _NBUF, batch, accs)
        nfull = (_NT // _NBUF) * _NBUF
        for s in range(_NT - nfull):
            tile_copy(nfull + s, s).wait()

            def inner_t(j, a, s=s):
                return tuple(
                    a[rr] + jnp.maximum(
                        buf3[s, rr, pl.ds(j * 16, 16)] - ts[rr], 0.0)
                    for rr in range(8))

            accs = lax.fori_loop(0, 8, inner_t, accs)

        for rr in range(8):
            s_r = jnp.sum(accs[rr])
            res = jnp.where(lane == g * 8 + rr, s_r, res)
    res_v[...] = res
    oyr_v[...] = oyr
    pltpu.sync_copy(res_v.at[pl.ds(0, _RPW)],
                    shared_sp.at[pl.ds(sid * _RPW, _RPW)])
    pltpu.sync_copy(oyr_v.at[pl.ds(0, _RPW)],
                    shared_oy.at[pl.ds(sid * _RPW, _RPW)])
    plsc.subcore_barrier()

    @pl.when(sid == 0)
    def _():
        half = _SC_ROWS // _NC
        off = pl.multiple_of(cid * half, 8)
        pltpu.sync_copy(shared_sp, sp_hbm.at[pl.ds(off, half)])
        pltpu.sync_copy(shared_oy, oyf_hbm.at[pl.ds(off, half)])


def _sc_partial(output, y1, oy_tail):
    mesh = plsc.VectorSubcoreMesh(core_axis_name="c", subcore_axis_name="s")
    f = pl.kernel(
        _sc_body,
        out_type=(jax.ShapeDtypeStruct((_SC_ROWS,), jnp.float32),
                  jax.ShapeDtypeStruct((_SC_ROWS,), jnp.float32)),
        mesh=mesh,
        scratch_types=[
            pltpu.VMEM((_B,), jnp.int32),
            pltpu.VMEM((_SC_ROWS,), jnp.float32),
            pltpu.VMEM((8, 128), jnp.float32),
            pltpu.VMEM((_NBUF, 8, 128), jnp.float32),
            pltpu.VMEM((16,), jnp.float32),
            pltpu.VMEM((16,), jnp.float32),
            pltpu.VMEM_SHARED((_SC_ROWS // _NC,), jnp.float32),
            pltpu.VMEM_SHARED((_SC_ROWS // _NC,), jnp.float32),
            pltpu.SemaphoreType.DMA((_NBUF,)),
        ],
        compiler_params=pltpu.CompilerParams(needs_layout_passes=False),
    )
    return f(output, y1, oy_tail)


def kernel(output, y):
    b, c = output.shape
    y1 = y.astype(jnp.int32)
    y2 = y1.reshape(b, 1)

    loss_tc = pl.pallas_call(
        _tc_body,
        grid=(_B_TC // _BR,),
        in_specs=[
            pl.BlockSpec((_BR, 1), lambda i: (i, 0)),
            pl.BlockSpec((_BR, c), lambda i: (i, 0)),
        ],
        out_specs=pl.BlockSpec((_BR, 1), lambda i: (i, 0)),
        out_shape=jax.ShapeDtypeStruct((_B_TC, 1), jnp.float32),
    )(y2, output)

    x32 = lax.slice(output, (_B_TC, _TAIL0), (_B, _C))

    oy_tail = pl.pallas_call(
        _tailgather_body,
        grid=(1,),
        in_specs=[
            pl.BlockSpec((_SC_ROWS, 1), lambda i: (3, 0)),
            pl.BlockSpec((_SC_ROWS, _TAIL_W), lambda i: (0, 0)),
        ],
        out_specs=pl.BlockSpec((_SC_ROWS, 1), lambda i: (0, 0)),
        out_shape=jax.ShapeDtypeStruct((_SC_ROWS, 1), jnp.float32),
    )(y2, x32)

    sp, oyf = _sc_partial(output, y1, oy_tail.reshape(_SC_ROWS))

    loss_tail = pl.pallas_call(
        _tail_body,
        grid=(_SC_ROWS // 64,),
        in_specs=[
            pl.BlockSpec((64, 1), lambda i: (i, 0)),
            pl.BlockSpec((64, 1), lambda i: (i, 0)),
            pl.BlockSpec((64, _TAIL_W), lambda i: (i, 0)),
        ],
        out_specs=pl.BlockSpec((64, 1), lambda i: (i, 0)),
        out_shape=jax.ShapeDtypeStruct((_SC_ROWS, 1), jnp.float32),
    )(oyf.reshape(_SC_ROWS, 1), sp.reshape(_SC_ROWS, 1), x32)

    return jnp.concatenate([loss_tc.reshape(_B_TC),
                            loss_tail.reshape(_SC_ROWS)])
